# baseline (speedup 1.0000x reference)
import functools

import jax
import jax.numpy as jnp
from jax import lax
from jax.experimental import pallas as pl
from jax.experimental.pallas import tpu as pltpu
from jax.experimental.pallas import tpu_sc as plsc

N = 10000
E = 320000
D = 128

NC = 2
NS = 16
CH = 32
CHUNKS_LO = (E // CH) // (NC * NS)
HEAVY = (E // CH) - CHUNKS_LO * NC * NS
EPT_MAX = CH * (CHUNKS_LO + 1)
ROWS_PER_TILE = 632
N_PAD = ROWS_PER_TILE * NS
ROW_CHUNKS = (CH,) * (ROWS_PER_TILE // CH) + (
    (ROWS_PER_TILE % CH,) if ROWS_PER_TILE % CH else ())


def _mm_body(x_ref, w_ref, o_ref):
    o_ref[...] = jnp.dot(x_ref[...], w_ref[...],
                         preferred_element_type=jnp.float32)


def _matmul(x, W):
    return pl.pallas_call(
        _mm_body,
        grid=(5,),
        in_specs=[
            pl.BlockSpec((N // 5, D), lambda i: (i, 0)),
            pl.BlockSpec((D, D), lambda i: (0, 0)),
        ],
        out_specs=pl.BlockSpec((N // 5, D), lambda i: (i, 0)),
        out_shape=jax.ShapeDtypeStruct((N, D), jnp.float32),
    )(x, W)


def _bcast16(vec, j):
    return lax.gather(
        vec, jnp.full((16, 1), j, jnp.int32),
        lax.GatherDimensionNumbers(
            offset_dims=(), collapsed_slice_dims=(0,),
            start_index_map=(0,)),
        slice_sizes=(1,),
        mode=lax.GatherScatterMode.PROMISE_IN_BOUNDS)


def _scale_rows(buf, ev1, c):
    def gbody(g, carry):
        evg = ev1[pl.ds(c * CH + g * 16, 16)]
        for j in range(16):
            sc = _bcast16(evg, j)
            e = g * 16 + j
            for h in range(D // 16):
                buf[e, pl.ds(h * 16, 16)] = buf[e, pl.ds(h * 16, 16)] * sc
        return carry

    lax.fori_loop(0, CH // 16, gbody, 0)


def _sc_body(support_hbm, eidx_hbm, ev_hbm, out_hbm,
             colb0, colb1, colb2, colb3, colb4, colb5, colb6, colb7,
             rowb0, rowb1, rowb2, rowb3, rowb4, rowb5, rowb6, rowb7,
             ev1, bufA, bufB, bufC, bufD, acc,
             semI0, semI1, semI2, semI3, semI4, semI5, semI6, semI7,
             semG0, semG1, semG2, semG3, semS0, semS1, semS2, semS3):
    c_ax = lax.axis_index("c")
    s = lax.axis_index("s")
    wid = c_ax * NS + s
    cols = (colb0, colb1, colb2, colb3, colb4, colb5, colb6, colb7)
    rows_ = (rowb0, rowb1, rowb2, rowb3, rowb4, rowb5, rowb6, rowb7)
    bufs = (bufA, bufB, bufC, bufD)
    semI = (semI0, semI1, semI2, semI3, semI4, semI5, semI6, semI7)
    semG = (semG0, semG1, semG2, semG3)
    semS = (semS0, semS1, semS2, semS3)
    base0 = CH * (CHUNKS_LO * wid + jnp.minimum(wid, HEAVY))
    T = lax.select(wid < HEAVY, CHUNKS_LO + 1, CHUNKS_LO)

    def idx_issue(j, m):
        pltpu.async_copy(eidx_hbm.at[1, pl.ds(base0 + j * CH, CH)],
                         cols[m], semI[m])
        pltpu.async_copy(eidx_hbm.at[0, pl.ds(base0 + j * CH, CH)],
                         rows_[m], semI[m])

    def idx_wait(m):
        pltpu.make_async_copy(eidx_hbm.at[1, pl.ds(base0, CH)],
                              cols[m], semI[m]).wait()
        pltpu.make_async_copy(eidx_hbm.at[0, pl.ds(base0, CH)],
                              rows_[m], semI[m]).wait()

    zero = jnp.zeros((16,), jnp.float32)

    def zbody(i, carry):
        for h in range(D // 16):
            bufA[i, pl.ds(h * 16, 16)] = zero
        return carry

    lax.fori_loop(0, CH, zbody, 0)
    off = 0
    for nrows in ROW_CHUNKS:
        pltpu.sync_copy(bufA.at[pl.ds(0, nrows)],
                        acc.at[pl.ds(s * ROWS_PER_TILE + off, nrows)])
        off += nrows

    @pl.when(wid < HEAVY)
    def _():
        pltpu.sync_copy(ev_hbm.at[pl.ds(base0, EPT_MAX)], ev1)

    @pl.when(wid >= HEAVY)
    def _():
        pltpu.sync_copy(ev_hbm.at[pl.ds(base0, CH * CHUNKS_LO)],
                        ev1.at[pl.ds(0, CH * CHUNKS_LO)])

    plsc.subcore_barrier()
    for j in range(6):
        idx_issue(j, j)
    idx_wait(0)
    pltpu.async_copy(support_hbm.at[cols[0]], bufs[0], semG[0])
    idx_wait(1)
    pltpu.async_copy(support_hbm.at[cols[1]], bufs[1], semG[1])

    def body(k, carry):
        for u in range(8):
            c = k * 8 + u
            mW = (u + 6) % 8
            bW = (u + 2) % 4

            @pl.when(jnp.logical_and(c >= 2, c - 2 < T))
            def _():
                pltpu.make_async_copy(bufs[bW], acc.at[rows_[mW]],
                                      semS[bW]).wait()

            @pl.when(c + 6 < T)
            def _():
                idx_issue(c + 6, mW)

            @pl.when(c + 2 < T)
            def _():
                idx_wait((u + 2) % 8)
                pltpu.async_copy(support_hbm.at[cols[(u + 2) % 8]],
                                 bufs[bW], semG[bW])

            @pl.when(c < T)
            def _():
                pltpu.make_async_copy(support_hbm.at[cols[u]],
                                      bufs[u % 4], semG[u % 4]).wait()
                _scale_rows(bufs[u % 4], ev1, c)
                pltpu.async_copy(bufs[u % 4], acc.at[rows_[u]],
                                 semS[u % 4], add=True)
        return carry

    lax.fori_loop(0, (T + 9) // 8, body, 0)
    plsc.subcore_barrier()

    off = 0
    for nrows in ROW_CHUNKS:
        r0 = s * ROWS_PER_TILE + off
        pltpu.sync_copy(acc.at[pl.ds(r0, nrows)],
                        out_hbm.at[c_ax, pl.ds(r0, nrows)])
        off += nrows


_sc_scatter = functools.partial(
    pl.kernel,
    out_type=jax.ShapeDtypeStruct((NC, N_PAD, D), jnp.float32),
    mesh=plsc.VectorSubcoreMesh(core_axis_name="c", subcore_axis_name="s"),
    scratch_types=(
        [pltpu.VMEM((CH,), jnp.int32)] * 8
        + [pltpu.VMEM((CH,), jnp.int32)] * 8
        + [pltpu.VMEM((EPT_MAX,), jnp.float32)]
        + [pltpu.VMEM((CH, D), jnp.float32)] * 4
        + [pltpu.VMEM_SHARED((N_PAD, D), jnp.float32)]
        + [pltpu.SemaphoreType.DMA] * 8
        + [pltpu.SemaphoreType.DMA] * 4
        + [pltpu.SemaphoreType.DMA] * 4
    ),
)(_sc_body)


def _comb_body(p_ref, b_ref, o_ref):
    o_ref[...] = p_ref[0] + p_ref[1] + b_ref[...]


def _combine(parts, b):
    return pl.pallas_call(
        _comb_body,
        grid=(10,),
        in_specs=[
            pl.BlockSpec((NC, N // 10, D), lambda i: (0, i, 0)),
            pl.BlockSpec((1, D), lambda i: (0, 0)),
        ],
        out_specs=pl.BlockSpec((N // 10, D), lambda i: (i, 0)),
        out_shape=jax.ShapeDtypeStruct((N, D), jnp.float32),
    )(parts, b.reshape(1, D))


def kernel(x, edge_index, edge_values, W, b):
    support = _matmul(x, W)
    parts = _sc_scatter(support, edge_index, edge_values)
    return _combine(parts, b)


# speedup vs baseline: 1.1313x; 1.1313x over previous
import functools

import jax
import jax.numpy as jnp
from jax import lax
from jax.experimental import pallas as pl
from jax.experimental.pallas import tpu as pltpu
from jax.experimental.pallas import tpu_sc as plsc

N = 10000
E = 320000
D = 128

NC = 2
NS = 16
CH = 64
CHUNKS_LO = (E // CH) // (NC * NS)
HEAVY = (E // CH) - CHUNKS_LO * NC * NS
EPT_MAX = CH * (CHUNKS_LO + 1)
ROWS_PER_TILE = 632
N_PAD = ROWS_PER_TILE * NS
ROW_CHUNKS = (64,) * 9 + (56,)


def _mm_body(x_ref, w_ref, o_ref):
    o_ref[...] = jnp.dot(x_ref[...], w_ref[...],
                         preferred_element_type=jnp.float32)


def _matmul(x, W):
    return pl.pallas_call(
        _mm_body,
        grid=(5,),
        in_specs=[
            pl.BlockSpec((N // 5, D), lambda i: (i, 0)),
            pl.BlockSpec((D, D), lambda i: (0, 0)),
        ],
        out_specs=pl.BlockSpec((N // 5, D), lambda i: (i, 0)),
        out_shape=jax.ShapeDtypeStruct((N, D), jnp.float32),
    )(x, W)


def _bcast16(vec, j):
    return lax.gather(
        vec, jnp.full((16, 1), j, jnp.int32),
        lax.GatherDimensionNumbers(
            offset_dims=(), collapsed_slice_dims=(0,),
            start_index_map=(0,)),
        slice_sizes=(1,),
        mode=lax.GatherScatterMode.PROMISE_IN_BOUNDS)


def _scale_rows(buf, ev1, c):
    def gbody(g, carry):
        evg = ev1[pl.ds(c * CH + g * 16, 16)]
        for j in range(16):
            sc = _bcast16(evg, j)
            e = g * 16 + j
            for h in range(D // 16):
                buf[e, pl.ds(h * 16, 16)] = buf[e, pl.ds(h * 16, 16)] * sc
        return carry

    lax.fori_loop(0, CH // 16, gbody, 0)


def _sc_body(support_hbm, eidx_hbm, ev_hbm, out_hbm,
             colb0, colb1, colb2, colb3, colb4, colb5, colb6, colb7,
             rowb0, rowb1, rowb2, rowb3, rowb4, rowb5, rowb6, rowb7,
             ev1, bufA, bufB, bufC, bufD, acc,
             semI0, semI1, semI2, semI3, semI4, semI5, semI6, semI7,
             semG0, semG1, semG2, semG3, semS0, semS1, semS2, semS3):
    c_ax = lax.axis_index("c")
    s = lax.axis_index("s")
    wid = c_ax * NS + s
    cols = (colb0, colb1, colb2, colb3, colb4, colb5, colb6, colb7)
    rows_ = (rowb0, rowb1, rowb2, rowb3, rowb4, rowb5, rowb6, rowb7)
    bufs = (bufA, bufB, bufC, bufD)
    semI = (semI0, semI1, semI2, semI3, semI4, semI5, semI6, semI7)
    semG = (semG0, semG1, semG2, semG3)
    semS = (semS0, semS1, semS2, semS3)
    base0 = CH * (CHUNKS_LO * wid + jnp.minimum(wid, HEAVY))
    T = lax.select(wid < HEAVY, CHUNKS_LO + 1, CHUNKS_LO)

    def idx_issue(j, m):
        pltpu.async_copy(eidx_hbm.at[1, pl.ds(base0 + j * CH, CH)],
                         cols[m], semI[m])
        pltpu.async_copy(eidx_hbm.at[0, pl.ds(base0 + j * CH, CH)],
                         rows_[m], semI[m])

    def idx_wait(m):
        pltpu.make_async_copy(eidx_hbm.at[1, pl.ds(base0, CH)],
                              cols[m], semI[m]).wait()
        pltpu.make_async_copy(eidx_hbm.at[0, pl.ds(base0, CH)],
                              rows_[m], semI[m]).wait()

    zero = jnp.zeros((16,), jnp.float32)

    def zbody(i, carry):
        for h in range(D // 16):
            bufA[i, pl.ds(h * 16, 16)] = zero
        return carry

    lax.fori_loop(0, CH, zbody, 0)
    off = 0
    for nrows in ROW_CHUNKS:
        pltpu.sync_copy(bufA.at[pl.ds(0, nrows)],
                        acc.at[pl.ds(s * ROWS_PER_TILE + off, nrows)])
        off += nrows

    @pl.when(wid < HEAVY)
    def _():
        pltpu.sync_copy(ev_hbm.at[pl.ds(base0, EPT_MAX)], ev1)

    @pl.when(wid >= HEAVY)
    def _():
        pltpu.sync_copy(ev_hbm.at[pl.ds(base0, CH * CHUNKS_LO)],
                        ev1.at[pl.ds(0, CH * CHUNKS_LO)])

    plsc.subcore_barrier()
    for j in range(6):
        idx_issue(j, j)
    idx_wait(0)
    pltpu.async_copy(support_hbm.at[cols[0]], bufs[0], semG[0])
    idx_wait(1)
    pltpu.async_copy(support_hbm.at[cols[1]], bufs[1], semG[1])

    def body(k, carry):
        for u in range(8):
            c = k * 8 + u
            mW = (u + 6) % 8
            bW = (u + 2) % 4

            @pl.when(jnp.logical_and(c >= 2, c - 2 < T))
            def _():
                pltpu.make_async_copy(bufs[bW], acc.at[rows_[mW]],
                                      semS[bW]).wait()

            @pl.when(c + 6 < T)
            def _():
                idx_issue(c + 6, mW)

            @pl.when(c + 2 < T)
            def _():
                idx_wait((u + 2) % 8)
                pltpu.async_copy(support_hbm.at[cols[(u + 2) % 8]],
                                 bufs[bW], semG[bW])

            @pl.when(c < T)
            def _():
                pltpu.make_async_copy(support_hbm.at[cols[u]],
                                      bufs[u % 4], semG[u % 4]).wait()
                _scale_rows(bufs[u % 4], ev1, c)
                pltpu.async_copy(bufs[u % 4], acc.at[rows_[u]],
                                 semS[u % 4], add=True)
        return carry

    lax.fori_loop(0, (T + 9) // 8, body, 0)
    plsc.subcore_barrier()

    off = 0
    for nrows in ROW_CHUNKS:
        r0 = s * ROWS_PER_TILE + off
        pltpu.sync_copy(acc.at[pl.ds(r0, nrows)],
                        out_hbm.at[c_ax, pl.ds(r0, nrows)])
        off += nrows


_sc_scatter = functools.partial(
    pl.kernel,
    out_type=jax.ShapeDtypeStruct((NC, N_PAD, D), jnp.float32),
    mesh=plsc.VectorSubcoreMesh(core_axis_name="c", subcore_axis_name="s"),
    scratch_types=(
        [pltpu.VMEM((CH,), jnp.int32)] * 8
        + [pltpu.VMEM((CH,), jnp.int32)] * 8
        + [pltpu.VMEM((EPT_MAX,), jnp.float32)]
        + [pltpu.VMEM((CH, D), jnp.float32)] * 4
        + [pltpu.VMEM_SHARED((N_PAD, D), jnp.float32)]
        + [pltpu.SemaphoreType.DMA] * 8
        + [pltpu.SemaphoreType.DMA] * 4
        + [pltpu.SemaphoreType.DMA] * 4
    ),
)(_sc_body)


def _comb_body(p_ref, b_ref, o_ref):
    o_ref[...] = p_ref[0] + p_ref[1] + b_ref[...]


def _combine(parts, b):
    return pl.pallas_call(
        _comb_body,
        grid=(10,),
        in_specs=[
            pl.BlockSpec((NC, N // 10, D), lambda i: (0, i, 0)),
            pl.BlockSpec((1, D), lambda i: (0, 0)),
        ],
        out_specs=pl.BlockSpec((N // 10, D), lambda i: (i, 0)),
        out_shape=jax.ShapeDtypeStruct((N, D), jnp.float32),
    )(parts, b.reshape(1, D))


def kernel(x, edge_index, edge_values, W, b):
    support = _matmul(x, W)
    parts = _sc_scatter(support, edge_index, edge_values)
    return _combine(parts, b)
